# Initial kernel scaffold; baseline (speedup 1.0000x reference)
import jax
import jax.numpy as jnp
from jax.experimental import pallas as pl


def kernel(x, inds, w, b):
    raise NotImplementedError("write your pallas kernel here")


# speedup vs baseline: 3.9887x; 3.9887x over previous
import jax
import jax.numpy as jnp
from jax.experimental import pallas as pl
from jax.experimental.pallas import tpu as pltpu


def _masked_body(inds_ref, x_ref, w_ref, b_ref, out_ref):
    c = pl.program_id(0)
    mask = inds_ref[:] == c
    xm = jnp.where(mask, x_ref[:], 0.0)
    contrib = jnp.dot(xm, w_ref[0], preferred_element_type=jnp.float32)
    contrib = contrib + jnp.where(mask, b_ref[0], 0.0)

    @pl.when(c == 0)
    def _init():
        out_ref[:] = contrib

    @pl.when(c > 0)
    def _acc():
        out_ref[:] += contrib


def kernel(x, inds, w, b):
    T, M = x.shape
    C, _, N = w.shape
    inds2 = inds.astype(jnp.int32).reshape(T, 1)

    out = pl.pallas_call(
        _masked_body,
        grid=(C,),
        in_specs=[
            pl.BlockSpec((T, 1), lambda c: (0, 0)),
            pl.BlockSpec((T, M), lambda c: (0, 0)),
            pl.BlockSpec((1, M, N), lambda c: (c, 0, 0)),
            pl.BlockSpec((1, 1, N), lambda c: (c, 0, 0)),
        ],
        out_specs=pl.BlockSpec((T, N), lambda c: (0, 0)),
        out_shape=jax.ShapeDtypeStruct((T, N), jnp.float32),
        compiler_params=pltpu.CompilerParams(
            dimension_semantics=("arbitrary",),
        ),
    )(inds2, x, w, b)
    return out
